# baseline (speedup 1.0000x reference)
import jax
import jax.numpy as jnp
from jax.experimental import pallas as pl
from jax.experimental.pallas import tpu as pltpu

_N_STRIPES = 8


def _copy_body(x_ref, o_ref, sems):
    rows = x_ref.shape[0]
    stripe = rows // _N_STRIPES
    copies = []
    for i in range(_N_STRIPES):
        c = pltpu.make_async_copy(
            x_ref.at[pl.ds(i * stripe, stripe), :],
            o_ref.at[pl.ds(i * stripe, stripe), :],
            sems.at[i],
        )
        c.start()
        copies.append(c)
    for c in copies:
        c.wait()


def kernel(x):
    return pl.pallas_call(
        _copy_body,
        in_specs=[pl.BlockSpec(memory_space=pl.ANY)],
        out_specs=pl.BlockSpec(memory_space=pl.ANY),
        out_shape=jax.ShapeDtypeStruct(x.shape, x.dtype),
        scratch_shapes=[pltpu.SemaphoreType.DMA((_N_STRIPES,))],
    )(x)


# speedup vs baseline: 49.0670x; 49.0670x over previous
import jax
import jax.numpy as jnp
from jax.experimental import pallas as pl
from jax.experimental.pallas import tpu as pltpu

_BM = 2048


def _copy_body(x_ref, o_ref):
    o_ref[...] = x_ref[...]


def kernel(x):
    m, n = x.shape
    return pl.pallas_call(
        _copy_body,
        grid=(m // _BM,),
        in_specs=[pl.BlockSpec((_BM, n), lambda i: (i, 0))],
        out_specs=pl.BlockSpec((_BM, n), lambda i: (i, 0)),
        out_shape=jax.ShapeDtypeStruct((m, n), x.dtype),
    )(x)
